# Initial kernel scaffold; baseline (speedup 1.0000x reference)
import jax
import jax.numpy as jnp
from jax.experimental import pallas as pl


def kernel(src_x, src_xyz, xyz, W1, g1, b1, W2, gl, bl):
    raise NotImplementedError("write your pallas kernel here")


# speedup vs baseline: 12.0171x; 12.0171x over previous
import jax
import jax.numpy as jnp
from jax import lax
from jax.experimental import pallas as pl
from jax.experimental.pallas import tpu as pltpu
from jax.experimental.pallas import tpu_sc as plsc

B, N, M, K = 4, 8192, 2048, 32
Q = B * M
CIN = 16
CPAD = 32
RADIUS = 0.2
R2 = RADIUS * RADIUS
EPS = 1e-5
COUT1, COUT2 = 32, 64

NC, NS, LANES = 2, 16, 16
NW = NC * NS
QPW = Q // NW
NCH = N // LANES
GCH = 128
NGC = (K * QPW) // GCH


def _bf16_round(x):
    u = plsc.bitcast(x, jnp.int32)
    u = (u + 0x7FFF + ((u >> 16) & 1)) & jnp.int32(~0xFFFF)
    return plsc.bitcast(u, jnp.float32)


def _sc_body(q_hbm, src_hbm, srcx_hbm,
             x_hbm,
             sxr, syr, szr, qxr, qyr, qzr,
             sxb, syb, szb, sqs,
             idxbuf, gidx, fstage, sem):
    w = lax.axis_index("s") * NC + lax.axis_index("c")
    qbase = w * QPW
    b = qbase // M

    pltpu.sync_copy(src_hbm.at[pl.ds((b * 3 + 0) * N, N)], sxr)
    pltpu.sync_copy(src_hbm.at[pl.ds((b * 3 + 1) * N, N)], syr)
    pltpu.sync_copy(src_hbm.at[pl.ds((b * 3 + 2) * N, N)], szr)
    pltpu.sync_copy(q_hbm.at[pl.ds(0 * Q + qbase, QPW)], qxr)
    pltpu.sync_copy(q_hbm.at[pl.ds(1 * Q + qbase, QPW)], qyr)
    pltpu.sync_copy(q_hbm.at[pl.ds(2 * Q + qbase, QPW)], qzr)

    lane = lax.iota(jnp.int32, LANES)
    bN = b * N

    def prep(c, carry):
        sl = pl.ds(c * LANES, LANES)
        sx, sy, sz = sxr[sl], syr[sl], szr[sl]
        sxb[sl] = _bf16_round(sx)
        syb[sl] = _bf16_round(sy)
        szb[sl] = _bf16_round(sz)
        sqs[sl] = (sx * sx + sy * sy) + sz * sz
        return carry

    lax.fori_loop(0, NCH, prep, 0)

    def per_query(i, carry):
        iv = jnp.full((LANES,), 0, jnp.int32) + i
        qxv = plsc.load_gather(qxr, [iv])
        qyv = plsc.load_gather(qyr, [iv])
        qzv = plsc.load_gather(qzr, [iv])
        qxb = _bf16_round(qxv)
        qyb = _bf16_round(qyv)
        qzb = _bf16_round(qzv)
        sqq = (qxv * qxv + qyv * qyv) + qzv * qzv
        ibase = i * K

        def cond(st):
            c, cnt, _ = st
            return (c < NCH) & (cnt < K)

        def body(st):
            c, cnt, first = st
            sl = pl.ds(c * LANES, LANES)
            t = qxb * sxb[sl] + qyb * syb[sl] + qzb * szb[sl]
            d2 = (sqq + sqs[sl]) - 2.0 * t
            m = d2 < R2
            mi = m.astype(jnp.int32)
            pc = jnp.sum(mi)
            pos = plsc.cumsum(mi)
            idxv = c * LANES + lane
            slot = ibase + cnt + pos - 1
            take = m & ((cnt + pos) <= K)
            plsc.store_scatter(idxbuf, [slot], idxv, mask=take)
            fcand = jnp.min(jnp.where(m, idxv, jnp.int32(N)))
            first = jnp.where((cnt == 0) & (pc > 0), fcand, first)
            cnt = jnp.minimum(cnt + pc, K)
            return (c + 1, cnt, first)

        z = jnp.int32(0)
        _, cnt, first = lax.while_loop(cond, body, (z, z, z))

        firstv = jnp.full((LANES,), 0, jnp.int32) + first
        for j in range(K // LANES):
            sl = pl.ds(ibase + j * LANES, LANES)
            kv = j * LANES + lane
            gi = jnp.where(kv < cnt, idxbuf[sl], firstv)
            idxbuf[sl] = gi
            flat = kv * QPW + i
            plsc.store_scatter(gidx, [flat >> 7, flat & (GCH - 1)], gi + bN)
        return carry

    lax.fori_loop(0, QPW, per_query, 0)

    nq = QPW // GCH

    def out_chunk(r, carry):
        k_r = r // nq
        qoff = (r % nq) * GCH
        pltpu.async_copy(srcx_hbm.at[gidx.at[r]], fstage, sem).wait()
        for j in range(GCH // LANES):
            rowv = j * LANES + lane
            gi = gidx[r, pl.ds(j * LANES, LANES)] - bN
            qsl = pl.ds(qoff + j * LANES, LANES)
            cx = plsc.load_gather(sxr, [gi]) - qxr[qsl]
            cy = plsc.load_gather(syr, [gi]) - qyr[qsl]
            cz = plsc.load_gather(szr, [gi]) - qzr[qsl]
            plsc.store_scatter(fstage, [rowv, jnp.full((LANES,), CIN, jnp.int32)], cx)
            plsc.store_scatter(fstage, [rowv, jnp.full((LANES,), CIN + 1, jnp.int32)], cy)
            plsc.store_scatter(fstage, [rowv, jnp.full((LANES,), CIN + 2, jnp.int32)], cz)
        pltpu.sync_copy(fstage, x_hbm.at[k_r, pl.ds(qbase + qoff, GCH)])
        return carry

    lax.fori_loop(0, NGC, out_chunk, 0)


def _sc_group(q_soa, src_soa, srcx32):
    mesh = plsc.VectorSubcoreMesh(core_axis_name="c", subcore_axis_name="s",
                                  num_cores=NC, num_subcores=NS)
    return pl.kernel(
        _sc_body,
        out_type=jax.ShapeDtypeStruct((K, Q, CPAD), jnp.float32),
        mesh=mesh,
        compiler_params=pltpu.CompilerParams(use_tc_tiling_on_sc=False,
                                             needs_layout_passes=False),
        scratch_types=[
            pltpu.VMEM((N,), jnp.float32),
            pltpu.VMEM((N,), jnp.float32),
            pltpu.VMEM((N,), jnp.float32),
            pltpu.VMEM((QPW,), jnp.float32),
            pltpu.VMEM((QPW,), jnp.float32),
            pltpu.VMEM((QPW,), jnp.float32),
            pltpu.VMEM((N,), jnp.float32),
            pltpu.VMEM((N,), jnp.float32),
            pltpu.VMEM((N,), jnp.float32),
            pltpu.VMEM((N,), jnp.float32),
            pltpu.VMEM((QPW * K,), jnp.int32),
            pltpu.VMEM((NGC, GCH), jnp.int32),
            pltpu.VMEM((GCH, CPAD), jnp.float32),
            pltpu.SemaphoreType.DMA,
        ],
    )(q_soa, src_soa, srcx32)


QB = 256
NQB = Q // QB


def _gelu(x):
    return 0.5 * x * (1.0 + lax.erf(x * jnp.float32(0.7071067811865475)))


def _t1_body(x_ref, w1p_ref, stats_ref):
    qb, k = pl.program_id(0), pl.program_id(1)
    y = lax.dot_general(x_ref[...].reshape(QB, CPAD), w1p_ref[...],
                        (((1,), (1,)), ((), ())),
                        preferred_element_type=jnp.float32)
    blk = jnp.stack([jnp.sum(y, 0), jnp.sum(y * y, 0)])

    @pl.when((qb == 0) & (k == 0))
    def _():
        stats_ref[...] = blk

    @pl.when((qb > 0) | (k > 0))
    def _():
        stats_ref[...] += blk


def _t2_body(x_ref, w1p_ref, a1_ref, c1_ref, w2_ref, zmax_ref, stats_ref):
    qb, k = pl.program_id(0), pl.program_id(1)
    y = lax.dot_general(x_ref[...].reshape(QB, CPAD), w1p_ref[...],
                        (((1,), (1,)), ((), ())),
                        preferred_element_type=jnp.float32)
    y = _gelu(y * a1_ref[...] + c1_ref[...])
    z = lax.dot_general(y, w2_ref[...], (((1,), (1,)), ((), ())),
                        preferred_element_type=jnp.float32)

    @pl.when(k == 0)
    def _():
        zmax_ref[...] = z

    @pl.when(k > 0)
    def _():
        zmax_ref[...] = jnp.maximum(zmax_ref[...], z)

    @pl.when(k == K - 1)
    def _():
        zm = zmax_ref[...]
        blk = jnp.stack([jnp.sum(zm, 0), jnp.sum(zm * zm, 0)])

        @pl.when(qb == 0)
        def _():
            stats_ref[...] = blk

        @pl.when(qb > 0)
        def _():
            stats_ref[...] += blk


def _t3_body(zmax_ref, a2_ref, c2_ref, out_ref):
    out_ref[...] = _gelu(zmax_ref[...] * a2_ref[...] + c2_ref[...])[None]


def kernel(src_x, src_xyz, xyz, W1, g1, b1, W2, gl, bl):
    q_soa = xyz.reshape(Q, 3).T.reshape(3 * Q)
    src_soa = src_xyz.transpose(0, 2, 1).reshape(B * 3 * N)
    srcx32 = jnp.concatenate(
        [src_x.reshape(B * N, CIN),
         jnp.zeros((B * N, CPAD - CIN), jnp.float32)], axis=1)
    w1p = jnp.concatenate(
        [W1[:, 3:], W1[:, :3],
         jnp.zeros((COUT1, CPAD - CIN - 3), jnp.float32)], axis=1)

    x_g = _sc_group(q_soa, src_soa, srcx32)

    x_spec = pl.BlockSpec((1, QB, CPAD), lambda qb, k: (k, qb, 0))
    full = lambda *shape: pl.BlockSpec(shape, lambda *_: (0,) * len(shape))

    stats1 = pl.pallas_call(
        _t1_body,
        grid=(NQB, K),
        in_specs=[x_spec, full(COUT1, CPAD)],
        out_specs=full(2, COUT1),
        out_shape=jax.ShapeDtypeStruct((2, COUT1), jnp.float32),
    )(x_g, w1p)

    cnt1 = jnp.float32(Q * K)
    mu1 = stats1[0] / cnt1
    var1 = stats1[1] / cnt1 - mu1 * mu1
    a1 = (g1 / jnp.sqrt(var1 + EPS)).reshape(1, COUT1)
    c1 = (b1 - mu1 * a1[0]).reshape(1, COUT1)

    zmax, stats2 = pl.pallas_call(
        _t2_body,
        grid=(NQB, K),
        in_specs=[x_spec, full(COUT1, CPAD),
                  full(1, COUT1), full(1, COUT1), full(COUT2, COUT1)],
        out_specs=[pl.BlockSpec((QB, COUT2), lambda qb, k: (qb, 0)),
                   full(2, COUT2)],
        out_shape=[jax.ShapeDtypeStruct((Q, COUT2), jnp.float32),
                   jax.ShapeDtypeStruct((2, COUT2), jnp.float32)],
    )(x_g, w1p, a1, c1, W2)

    cnt2 = jnp.float32(Q)
    mu2 = stats2[0] / cnt2
    var2 = stats2[1] / cnt2 - mu2 * mu2
    a2 = (gl / jnp.sqrt(var2 + EPS)).reshape(1, COUT2)
    c2 = (bl - mu2 * a2[0]).reshape(1, COUT2)

    out = pl.pallas_call(
        _t3_body,
        grid=(NQB,),
        in_specs=[pl.BlockSpec((QB, COUT2), lambda p: (p, 0)),
                  full(1, COUT2), full(1, COUT2)],
        out_specs=pl.BlockSpec((1, QB, COUT2),
                               lambda p: (p // (M // QB), p % (M // QB), 0)),
        out_shape=jax.ShapeDtypeStruct((B, M, COUT2), jnp.float32),
    )(zmax, a2, c2)
    return out


# speedup vs baseline: 24.8812x; 2.0705x over previous
import jax
import jax.numpy as jnp
from jax import lax
from jax.experimental import pallas as pl
from jax.experimental.pallas import tpu as pltpu
from jax.experimental.pallas import tpu_sc as plsc

B, N, M, K = 4, 8192, 2048, 32
Q = B * M
CIN = 16
CPAD = 32
RADIUS = 0.2
R2 = RADIUS * RADIUS
EPS = 1e-5
COUT1, COUT2 = 32, 64

NC, NS, LANES = 2, 16, 16
NW = NC * NS
QPW = Q // NW
NCH = N // LANES
GCH = 128
NGC = (K * QPW) // GCH


def _bf16_round(x):
    u = plsc.bitcast(x, jnp.int32)
    u = (u + 0x7FFF + ((u >> 16) & 1)) & jnp.int32(~0xFFFF)
    return plsc.bitcast(u, jnp.float32)


def _sc_body(q_hbm, src_hbm, srcx_hbm,
             x_hbm,
             sxr, syr, szr, qxr, qyr, qzr,
             sxb, syb, szb, sqs,
             idxbuf, gidx, fstage, sem):
    w = lax.axis_index("s") * NC + lax.axis_index("c")
    qbase = w * QPW
    b = qbase // M

    pltpu.sync_copy(src_hbm.at[pl.ds((b * 3 + 0) * N, N)], sxr)
    pltpu.sync_copy(src_hbm.at[pl.ds((b * 3 + 1) * N, N)], syr)
    pltpu.sync_copy(src_hbm.at[pl.ds((b * 3 + 2) * N, N)], szr)
    pltpu.sync_copy(q_hbm.at[pl.ds(0 * Q + qbase, QPW)], qxr)
    pltpu.sync_copy(q_hbm.at[pl.ds(1 * Q + qbase, QPW)], qyr)
    pltpu.sync_copy(q_hbm.at[pl.ds(2 * Q + qbase, QPW)], qzr)

    lane = lax.iota(jnp.int32, LANES)
    bN = b * N

    def prep(c, carry):
        sl = pl.ds(c * LANES, LANES)
        sx, sy, sz = sxr[sl], syr[sl], szr[sl]
        sxb[sl] = _bf16_round(sx)
        syb[sl] = _bf16_round(sy)
        szb[sl] = _bf16_round(sz)
        sqs[sl] = (sx * sx + sy * sy) + sz * sz
        return carry

    lax.fori_loop(0, NCH, prep, 0)

    def per_query(i, carry):
        iv = jnp.full((LANES,), 0, jnp.int32) + i
        qxv = plsc.load_gather(qxr, [iv])
        qyv = plsc.load_gather(qyr, [iv])
        qzv = plsc.load_gather(qzr, [iv])
        qxb = _bf16_round(qxv)
        qyb = _bf16_round(qyv)
        qzb = _bf16_round(qzv)
        sqq = (qxv * qxv + qyv * qyv) + qzv * qzv
        ibase = i * K

        def cond(st):
            c, cnt, _ = st
            return (c < NCH) & (cnt < K)

        def body(st):
            c, cnt, first = st
            sl = pl.ds(c * LANES, LANES)
            t = qxb * sxb[sl] + qyb * syb[sl] + qzb * szb[sl]
            d2 = (sqq + sqs[sl]) - 2.0 * t
            m = d2 < R2
            mi = m.astype(jnp.int32)
            pc = jnp.sum(mi)
            pos = plsc.cumsum(mi)
            idxv = c * LANES + lane
            slot = ibase + cnt + pos - 1
            take = m & ((cnt + pos) <= K)
            plsc.store_scatter(idxbuf, [slot], idxv, mask=take)
            fcand = jnp.min(jnp.where(m, idxv, jnp.int32(N)))
            first = jnp.where((cnt == 0) & (pc > 0), fcand, first)
            cnt = jnp.minimum(cnt + pc, K)
            return (c + 1, cnt, first)

        z = jnp.int32(0)
        _, cnt, first = lax.while_loop(cond, body, (z, z, z))

        firstv = jnp.full((LANES,), 0, jnp.int32) + first
        for j in range(K // LANES):
            sl = pl.ds(ibase + j * LANES, LANES)
            kv = j * LANES + lane
            gi = jnp.where(kv < cnt, idxbuf[sl], firstv)
            idxbuf[sl] = gi
            flat = kv * QPW + i
            plsc.store_scatter(gidx, [flat >> 7, flat & (GCH - 1)], gi + bN)
        return carry

    lax.fori_loop(0, QPW, per_query, 0)

    nq = QPW // GCH

    def out_chunk(r, carry):
        k_r = r // nq
        qoff = (r % nq) * GCH
        pltpu.async_copy(srcx_hbm.at[gidx.at[r]], fstage, sem).wait()
        for j in range(GCH // LANES):
            rowv = j * LANES + lane
            gi = gidx[r, pl.ds(j * LANES, LANES)] - bN
            qsl = pl.ds(qoff + j * LANES, LANES)
            cx = plsc.load_gather(sxr, [gi]) - qxr[qsl]
            cy = plsc.load_gather(syr, [gi]) - qyr[qsl]
            cz = plsc.load_gather(szr, [gi]) - qzr[qsl]
            plsc.store_scatter(fstage, [rowv, jnp.full((LANES,), CIN, jnp.int32)], cx)
            plsc.store_scatter(fstage, [rowv, jnp.full((LANES,), CIN + 1, jnp.int32)], cy)
            plsc.store_scatter(fstage, [rowv, jnp.full((LANES,), CIN + 2, jnp.int32)], cz)
        pltpu.sync_copy(fstage, x_hbm.at[k_r, pl.ds(qbase + qoff, GCH)])
        return carry

    lax.fori_loop(0, NGC, out_chunk, 0)


def _sc_group(q_soa, src_soa, srcx32):
    mesh = plsc.VectorSubcoreMesh(core_axis_name="c", subcore_axis_name="s",
                                  num_cores=NC, num_subcores=NS)
    return pl.kernel(
        _sc_body,
        out_type=jax.ShapeDtypeStruct((K, Q, CPAD), jnp.float32),
        mesh=mesh,
        compiler_params=pltpu.CompilerParams(use_tc_tiling_on_sc=False,
                                             needs_layout_passes=False),
        scratch_types=[
            pltpu.VMEM((N,), jnp.float32),
            pltpu.VMEM((N,), jnp.float32),
            pltpu.VMEM((N,), jnp.float32),
            pltpu.VMEM((QPW,), jnp.float32),
            pltpu.VMEM((QPW,), jnp.float32),
            pltpu.VMEM((QPW,), jnp.float32),
            pltpu.VMEM((N,), jnp.float32),
            pltpu.VMEM((N,), jnp.float32),
            pltpu.VMEM((N,), jnp.float32),
            pltpu.VMEM((N,), jnp.float32),
            pltpu.VMEM((QPW * K,), jnp.int32),
            pltpu.VMEM((NGC, GCH), jnp.int32),
            pltpu.VMEM((GCH, CPAD), jnp.float32),
            pltpu.SemaphoreType.DMA,
        ],
    )(q_soa, src_soa, srcx32)


def _gelu(x):
    return 0.5 * x * (1.0 + lax.erf(x * jnp.float32(0.7071067811865475)))


def _t1_body(x_ref, w1p_ref, stats_ref):
    k = pl.program_id(0)
    y = lax.dot_general(x_ref[...].reshape(Q, CPAD), w1p_ref[...],
                        (((1,), (1,)), ((), ())),
                        preferred_element_type=jnp.float32)
    blk = jnp.stack([jnp.sum(y, 0), jnp.sum(y * y, 0)])

    @pl.when(k == 0)
    def _():
        stats_ref[...] = blk

    @pl.when(k > 0)
    def _():
        stats_ref[...] += blk


def _t2_body(x_ref, w1p_ref, a1_ref, c1_ref, w2_ref, zmax_ref, stats_ref):
    k = pl.program_id(0)
    y = lax.dot_general(x_ref[...].reshape(Q, CPAD), w1p_ref[...],
                        (((1,), (1,)), ((), ())),
                        preferred_element_type=jnp.float32)
    y = _gelu(y * a1_ref[...] + c1_ref[...])
    z = lax.dot_general(y, w2_ref[...], (((1,), (1,)), ((), ())),
                        preferred_element_type=jnp.float32)

    @pl.when(k == 0)
    def _():
        zmax_ref[...] = z

    @pl.when(k > 0)
    def _():
        zmax_ref[...] = jnp.maximum(zmax_ref[...], z)

    @pl.when(k == K - 1)
    def _():
        zm = zmax_ref[...]
        stats_ref[...] = jnp.stack([jnp.sum(zm, 0), jnp.sum(zm * zm, 0)])


def _t3_body(zmax_ref, a2_ref, c2_ref, out_ref):
    out_ref[...] = _gelu(zmax_ref[...] * a2_ref[...] + c2_ref[...]).reshape(
        B, M, COUT2)


def kernel(src_x, src_xyz, xyz, W1, g1, b1, W2, gl, bl):
    q_soa = xyz.reshape(Q, 3).T.reshape(3 * Q)
    src_soa = src_xyz.transpose(0, 2, 1).reshape(B * 3 * N)
    srcx32 = jnp.concatenate(
        [src_x.reshape(B * N, CIN),
         jnp.zeros((B * N, CPAD - CIN), jnp.float32)], axis=1)
    w1p = jnp.concatenate(
        [W1[:, 3:], W1[:, :3],
         jnp.zeros((COUT1, CPAD - CIN - 3), jnp.float32)], axis=1)

    x_g = _sc_group(q_soa, src_soa, srcx32)

    x_spec = pl.BlockSpec((1, Q, CPAD), lambda k: (k, 0, 0))
    full = lambda *shape: pl.BlockSpec(shape, lambda *_: (0,) * len(shape))

    stats1 = pl.pallas_call(
        _t1_body,
        grid=(K,),
        in_specs=[x_spec, full(COUT1, CPAD)],
        out_specs=full(2, COUT1),
        out_shape=jax.ShapeDtypeStruct((2, COUT1), jnp.float32),
    )(x_g, w1p)

    cnt1 = jnp.float32(Q * K)
    mu1 = stats1[0] / cnt1
    var1 = stats1[1] / cnt1 - mu1 * mu1
    a1 = (g1 / jnp.sqrt(var1 + EPS)).reshape(1, COUT1)
    c1 = (b1 - mu1 * a1[0]).reshape(1, COUT1)

    zmax, stats2 = pl.pallas_call(
        _t2_body,
        grid=(K,),
        in_specs=[x_spec, full(COUT1, CPAD),
                  full(1, COUT1), full(1, COUT1), full(COUT2, COUT1)],
        out_specs=[full(Q, COUT2), full(2, COUT2)],
        out_shape=[jax.ShapeDtypeStruct((Q, COUT2), jnp.float32),
                   jax.ShapeDtypeStruct((2, COUT2), jnp.float32)],
    )(x_g, w1p, a1, c1, W2)

    cnt2 = jnp.float32(Q)
    mu2 = stats2[0] / cnt2
    var2 = stats2[1] / cnt2 - mu2 * mu2
    a2 = (gl / jnp.sqrt(var2 + EPS)).reshape(1, COUT2)
    c2 = (bl - mu2 * a2[0]).reshape(1, COUT2)

    out = pl.pallas_call(
        _t3_body,
        grid=(1,),
        in_specs=[full(Q, COUT2), full(1, COUT2), full(1, COUT2)],
        out_specs=full(B, M, COUT2),
        out_shape=jax.ShapeDtypeStruct((B, M, COUT2), jnp.float32),
    )(zmax, a2, c2)
    return out


# speedup vs baseline: 28.9146x; 1.1621x over previous
import jax
import jax.numpy as jnp
from jax import lax
from jax.experimental import pallas as pl
from jax.experimental.pallas import tpu as pltpu
from jax.experimental.pallas import tpu_sc as plsc

B, N, M, K = 4, 8192, 2048, 32
Q = B * M
CIN = 16
CPAD = 32
RADIUS = 0.2
R2 = RADIUS * RADIUS
EPS = 1e-5
COUT1, COUT2 = 32, 64

NC, NS, LANES = 2, 16, 16
NW = NC * NS
QPW = Q // NW
NCH = N // LANES
GCH = 128
NGC = (K * QPW) // GCH


def _bf16_round(x):
    u = plsc.bitcast(x, jnp.int32)
    u = (u + 0x7FFF + ((u >> 16) & 1)) & jnp.int32(~0xFFFF)
    return plsc.bitcast(u, jnp.float32)


def _sc_body(q_hbm, src_hbm, srcx_hbm,
             x_hbm,
             sxr, syr, szr, qxr, qyr, qzr,
             sxb, syb, szb, sqs,
             idxbuf, gidx, fstage, sem):
    w = lax.axis_index("s") * NC + lax.axis_index("c")
    qbase = w * QPW
    b = qbase // M

    pltpu.sync_copy(src_hbm.at[pl.ds((b * 3 + 0) * N, N)], sxr)
    pltpu.sync_copy(src_hbm.at[pl.ds((b * 3 + 1) * N, N)], syr)
    pltpu.sync_copy(src_hbm.at[pl.ds((b * 3 + 2) * N, N)], szr)
    pltpu.sync_copy(q_hbm.at[pl.ds(0 * Q + qbase, QPW)], qxr)
    pltpu.sync_copy(q_hbm.at[pl.ds(1 * Q + qbase, QPW)], qyr)
    pltpu.sync_copy(q_hbm.at[pl.ds(2 * Q + qbase, QPW)], qzr)

    lane = lax.iota(jnp.int32, LANES)
    bN = b * N

    def prep(c, carry):
        sl = pl.ds(c * LANES, LANES)
        sx, sy, sz = sxr[sl], syr[sl], szr[sl]
        sxb[sl] = _bf16_round(sx)
        syb[sl] = _bf16_round(sy)
        szb[sl] = _bf16_round(sz)
        sqs[sl] = (sx * sx + sy * sy) + sz * sz
        return carry

    lax.fori_loop(0, NCH, prep, 0)

    UNROLL = 4
    NGR = NCH // UNROLL

    def per_query(i, carry):
        iv = jnp.full((LANES,), 0, jnp.int32) + i
        qxv = plsc.load_gather(qxr, [iv])
        qyv = plsc.load_gather(qyr, [iv])
        qzv = plsc.load_gather(qzr, [iv])
        qxb = _bf16_round(qxv)
        qyb = _bf16_round(qyv)
        qzb = _bf16_round(qzv)
        sqq = (qxv * qxv + qyv * qyv) + qzv * qzv
        ibase_v = jnp.full((LANES,), 0, jnp.int32) + i * K + lane

        def cond(st):
            c, cnt = st
            return (c < NGR) & jnp.any(cnt < K)

        def body(st):
            c, cnt = st
            base = c * (LANES * UNROLL)
            for j in range(UNROLL):
                sl = pl.ds(base + j * LANES, LANES)
                t = qxb * sxb[sl] + qyb * syb[sl] + qzb * szb[sl]
                d2 = (sqq + sqs[sl]) - 2.0 * t
                m = d2 < R2
                pos = plsc.cumsum(m.astype(jnp.int32))
                newcnt = cnt + pos
                take = m & (newcnt <= K)
                slot = (ibase_v - lane) + cnt + pos - 1
                idxv = base + j * LANES + lane
                plsc.store_scatter(idxbuf, [slot], idxv, mask=take)
                cnt = cnt + plsc.all_reduce_population_count(m)
            return (c + 1, jnp.minimum(cnt, K))

        z = jnp.int32(0)
        zv = jnp.zeros((LANES,), jnp.int32)
        _, cntv = lax.while_loop(cond, body, (z, zv))

        first0 = plsc.load_gather(idxbuf, [ibase_v - lane])
        firstv = jnp.where(cntv > 0, first0, 0)
        for j in range(K // LANES):
            sl = pl.ds(i * K + j * LANES, LANES)
            kv = j * LANES + lane
            gi = jnp.where(kv < cntv, idxbuf[sl], firstv)
            idxbuf[sl] = gi
            flat = kv * QPW + i
            plsc.store_scatter(gidx, [flat >> 7, flat & (GCH - 1)], gi + bN)
        return carry

    lax.fori_loop(0, QPW, per_query, 0)

    nq = QPW // GCH

    def out_chunk(r, carry):
        k_r = r // nq
        qoff = (r % nq) * GCH
        pltpu.async_copy(srcx_hbm.at[gidx.at[r]], fstage, sem).wait()
        for j in range(GCH // LANES):
            rowv = j * LANES + lane
            gi = gidx[r, pl.ds(j * LANES, LANES)] - bN
            qsl = pl.ds(qoff + j * LANES, LANES)
            cx = plsc.load_gather(sxr, [gi]) - qxr[qsl]
            cy = plsc.load_gather(syr, [gi]) - qyr[qsl]
            cz = plsc.load_gather(szr, [gi]) - qzr[qsl]
            plsc.store_scatter(fstage, [rowv, jnp.full((LANES,), CIN, jnp.int32)], cx)
            plsc.store_scatter(fstage, [rowv, jnp.full((LANES,), CIN + 1, jnp.int32)], cy)
            plsc.store_scatter(fstage, [rowv, jnp.full((LANES,), CIN + 2, jnp.int32)], cz)
        pltpu.sync_copy(fstage, x_hbm.at[k_r, pl.ds(qbase + qoff, GCH)])
        return carry

    lax.fori_loop(0, NGC, out_chunk, 0)


def _sc_group(q_soa, src_soa, srcx32):
    mesh = plsc.VectorSubcoreMesh(core_axis_name="c", subcore_axis_name="s",
                                  num_cores=NC, num_subcores=NS)
    return pl.kernel(
        _sc_body,
        out_type=jax.ShapeDtypeStruct((K, Q, CPAD), jnp.float32),
        mesh=mesh,
        compiler_params=pltpu.CompilerParams(use_tc_tiling_on_sc=False,
                                             needs_layout_passes=False),
        scratch_types=[
            pltpu.VMEM((N,), jnp.float32),
            pltpu.VMEM((N,), jnp.float32),
            pltpu.VMEM((N,), jnp.float32),
            pltpu.VMEM((QPW,), jnp.float32),
            pltpu.VMEM((QPW,), jnp.float32),
            pltpu.VMEM((QPW,), jnp.float32),
            pltpu.VMEM((N,), jnp.float32),
            pltpu.VMEM((N,), jnp.float32),
            pltpu.VMEM((N,), jnp.float32),
            pltpu.VMEM((N,), jnp.float32),
            pltpu.VMEM((QPW * K,), jnp.int32),
            pltpu.VMEM((NGC, GCH), jnp.int32),
            pltpu.VMEM((GCH, CPAD), jnp.float32),
            pltpu.SemaphoreType.DMA,
        ],
    )(q_soa, src_soa, srcx32)


def _gelu(x):
    return 0.5 * x * (1.0 + lax.erf(x * jnp.float32(0.7071067811865475)))


def _t1_body(x_ref, w1p_ref, stats_ref):
    k = pl.program_id(0)
    y = lax.dot_general(x_ref[...].reshape(Q, CPAD), w1p_ref[...],
                        (((1,), (1,)), ((), ())),
                        preferred_element_type=jnp.float32)
    blk = jnp.stack([jnp.sum(y, 0), jnp.sum(y * y, 0)])

    @pl.when(k == 0)
    def _():
        stats_ref[...] = blk

    @pl.when(k > 0)
    def _():
        stats_ref[...] += blk


def _t2_body(x_ref, w1p_ref, a1_ref, c1_ref, w2_ref, zmax_ref, stats_ref):
    k = pl.program_id(0)
    y = lax.dot_general(x_ref[...].reshape(Q, CPAD), w1p_ref[...],
                        (((1,), (1,)), ((), ())),
                        preferred_element_type=jnp.float32)
    y = _gelu(y * a1_ref[...] + c1_ref[...])
    z = lax.dot_general(y, w2_ref[...], (((1,), (1,)), ((), ())),
                        preferred_element_type=jnp.float32)

    @pl.when(k == 0)
    def _():
        zmax_ref[...] = z

    @pl.when(k > 0)
    def _():
        zmax_ref[...] = jnp.maximum(zmax_ref[...], z)

    @pl.when(k == K - 1)
    def _():
        zm = zmax_ref[...]
        stats_ref[...] = jnp.stack([jnp.sum(zm, 0), jnp.sum(zm * zm, 0)])


def _t3_body(zmax_ref, a2_ref, c2_ref, out_ref):
    out_ref[...] = _gelu(zmax_ref[...] * a2_ref[...] + c2_ref[...]).reshape(
        B, M, COUT2)


def kernel(src_x, src_xyz, xyz, W1, g1, b1, W2, gl, bl):
    q_soa = xyz.reshape(Q, 3).T.reshape(3 * Q)
    src_soa = src_xyz.transpose(0, 2, 1).reshape(B * 3 * N)
    srcx32 = jnp.concatenate(
        [src_x.reshape(B * N, CIN),
         jnp.zeros((B * N, CPAD - CIN), jnp.float32)], axis=1)
    w1p = jnp.concatenate(
        [W1[:, 3:], W1[:, :3],
         jnp.zeros((COUT1, CPAD - CIN - 3), jnp.float32)], axis=1)

    x_g = _sc_group(q_soa, src_soa, srcx32)

    x_spec = pl.BlockSpec((1, Q, CPAD), lambda k: (k, 0, 0))
    full = lambda *shape: pl.BlockSpec(shape, lambda *_: (0,) * len(shape))

    stats1 = pl.pallas_call(
        _t1_body,
        grid=(K,),
        in_specs=[x_spec, full(COUT1, CPAD)],
        out_specs=full(2, COUT1),
        out_shape=jax.ShapeDtypeStruct((2, COUT1), jnp.float32),
    )(x_g, w1p)

    cnt1 = jnp.float32(Q * K)
    mu1 = stats1[0] / cnt1
    var1 = stats1[1] / cnt1 - mu1 * mu1
    a1 = (g1 / jnp.sqrt(var1 + EPS)).reshape(1, COUT1)
    c1 = (b1 - mu1 * a1[0]).reshape(1, COUT1)

    zmax, stats2 = pl.pallas_call(
        _t2_body,
        grid=(K,),
        in_specs=[x_spec, full(COUT1, CPAD),
                  full(1, COUT1), full(1, COUT1), full(COUT2, COUT1)],
        out_specs=[full(Q, COUT2), full(2, COUT2)],
        out_shape=[jax.ShapeDtypeStruct((Q, COUT2), jnp.float32),
                   jax.ShapeDtypeStruct((2, COUT2), jnp.float32)],
    )(x_g, w1p, a1, c1, W2)

    cnt2 = jnp.float32(Q)
    mu2 = stats2[0] / cnt2
    var2 = stats2[1] / cnt2 - mu2 * mu2
    a2 = (gl / jnp.sqrt(var2 + EPS)).reshape(1, COUT2)
    c2 = (bl - mu2 * a2[0]).reshape(1, COUT2)

    out = pl.pallas_call(
        _t3_body,
        grid=(1,),
        in_specs=[full(Q, COUT2), full(1, COUT2), full(1, COUT2)],
        out_specs=full(B, M, COUT2),
        out_shape=jax.ShapeDtypeStruct((B, M, COUT2), jnp.float32),
    )(zmax, a2, c2)
    return out


# speedup vs baseline: 30.8071x; 1.0654x over previous
import jax
import jax.numpy as jnp
from jax import lax
from jax.experimental import pallas as pl
from jax.experimental.pallas import tpu as pltpu
from jax.experimental.pallas import tpu_sc as plsc

B, N, M, K = 4, 8192, 2048, 32
Q = B * M
CIN = 16
CPAD = 32
RADIUS = 0.2
R2 = RADIUS * RADIUS
EPS = 1e-5
COUT1, COUT2 = 32, 64

NC, NS, LANES = 2, 16, 16
NW = NC * NS
QPW = Q // NW
NCH = N // LANES
GCH = 128
NGC = (K * QPW) // GCH


def _bf16_round(x):
    u = plsc.bitcast(x, jnp.int32)
    u = (u + 0x7FFF + ((u >> 16) & 1)) & jnp.int32(~0xFFFF)
    return plsc.bitcast(u, jnp.float32)


def _sc_body(q_hbm, src_hbm, srcx_hbm,
             x_hbm,
             sxr, syr, szr, qxr, qyr, qzr,
             sxb, syb, szb, sqs,
             idxbuf, gidx, fstage, fstage2, fshared, sem, sem2, sem3):
    s_id = lax.axis_index("s")
    c_id = lax.axis_index("c")
    w = c_id * NS + s_id
    qbase = w * QPW
    b = qbase // M
    bloc = b - 2 * c_id

    RPS = (2 * N) // NS
    stage_dst = pl.ds(s_id * RPS, RPS)
    stage_src = pl.ds(c_id * 2 * N + s_id * RPS, RPS)
    pltpu.async_copy(srcx_hbm.at[stage_src], fshared.at[stage_dst], sem3)

    pltpu.sync_copy(src_hbm.at[pl.ds((b * 3 + 0) * N, N)], sxr)
    pltpu.sync_copy(src_hbm.at[pl.ds((b * 3 + 1) * N, N)], syr)
    pltpu.sync_copy(src_hbm.at[pl.ds((b * 3 + 2) * N, N)], szr)
    pltpu.sync_copy(q_hbm.at[pl.ds(0 * Q + qbase, QPW)], qxr)
    pltpu.sync_copy(q_hbm.at[pl.ds(1 * Q + qbase, QPW)], qyr)
    pltpu.sync_copy(q_hbm.at[pl.ds(2 * Q + qbase, QPW)], qzr)

    lane = lax.iota(jnp.int32, LANES)
    bN = bloc * N

    def prep(c, carry):
        sl = pl.ds(c * LANES, LANES)
        sx, sy, sz = sxr[sl], syr[sl], szr[sl]
        sxb[sl] = _bf16_round(sx)
        syb[sl] = _bf16_round(sy)
        szb[sl] = _bf16_round(sz)
        sqs[sl] = (sx * sx + sy * sy) + sz * sz
        return carry

    lax.fori_loop(0, NCH, prep, 0)

    UNROLL = 4
    NGR = NCH // UNROLL

    def per_query(i, carry):
        iv = jnp.full((LANES,), 0, jnp.int32) + i
        qxv = plsc.load_gather(qxr, [iv])
        qyv = plsc.load_gather(qyr, [iv])
        qzv = plsc.load_gather(qzr, [iv])
        qxb = _bf16_round(qxv)
        qyb = _bf16_round(qyv)
        qzb = _bf16_round(qzv)
        sqq = (qxv * qxv + qyv * qyv) + qzv * qzv
        ibase_v = jnp.full((LANES,), 0, jnp.int32) + i * K + lane

        def cond(st):
            c, cnt = st
            return (c < NGR) & jnp.any(cnt < K)

        def body(st):
            c, cnt = st
            base = c * (LANES * UNROLL)
            for j in range(UNROLL):
                sl = pl.ds(base + j * LANES, LANES)
                t = qxb * sxb[sl] + qyb * syb[sl] + qzb * szb[sl]
                d2 = (sqq + sqs[sl]) - 2.0 * t
                m = d2 < R2
                pos = plsc.cumsum(m.astype(jnp.int32))
                newcnt = cnt + pos
                take = m & (newcnt <= K)
                slot = (ibase_v - lane) + cnt + pos - 1
                idxv = base + j * LANES + lane
                plsc.store_scatter(idxbuf, [slot], idxv, mask=take)
                cnt = cnt + plsc.all_reduce_population_count(m)
            return (c + 1, jnp.minimum(cnt, K))

        z = jnp.int32(0)
        zv = jnp.zeros((LANES,), jnp.int32)
        _, cntv = lax.while_loop(cond, body, (z, zv))

        first0 = plsc.load_gather(idxbuf, [ibase_v - lane])
        firstv = jnp.where(cntv > 0, first0, 0)
        for j in range(K // LANES):
            sl = pl.ds(i * K + j * LANES, LANES)
            kv = j * LANES + lane
            gi = jnp.where(kv < cntv, idxbuf[sl], firstv)
            idxbuf[sl] = gi
            flat = kv * QPW + i
            plsc.store_scatter(gidx, [flat >> 7, flat & (GCH - 1)], gi + bN)
        return carry

    lax.fori_loop(0, QPW, per_query, 0)

    nq = QPW // GCH
    HALF = NGC // 2

    def _compute_and_write(r, buf):
        k_r = r // nq
        qoff = (r % nq) * GCH
        for j in range(GCH // LANES):
            rowv = j * LANES + lane
            gi = gidx[r, pl.ds(j * LANES, LANES)] - bN
            qsl = pl.ds(qoff + j * LANES, LANES)
            cx = plsc.load_gather(sxr, [gi]) - qxr[qsl]
            cy = plsc.load_gather(syr, [gi]) - qyr[qsl]
            cz = plsc.load_gather(szr, [gi]) - qzr[qsl]
            plsc.store_scatter(buf, [rowv, jnp.full((LANES,), CIN, jnp.int32)], cx)
            plsc.store_scatter(buf, [rowv, jnp.full((LANES,), CIN + 1, jnp.int32)], cy)
            plsc.store_scatter(buf, [rowv, jnp.full((LANES,), CIN + 2, jnp.int32)], cz)
        pltpu.sync_copy(buf, x_hbm.at[k_r, pl.ds(qbase + qoff, GCH)])

    pltpu.make_async_copy(srcx_hbm.at[stage_src], fshared.at[stage_dst], sem3).wait()
    plsc.subcore_barrier()

    pltpu.async_copy(fshared.at[gidx.at[0]], fstage, sem)

    def out_group(g, carry):
        r0 = 2 * g
        r1 = r0 + 1
        pltpu.async_copy(fshared.at[gidx.at[r1]], fstage2, sem2)
        pltpu.make_async_copy(fshared.at[gidx.at[r0]], fstage, sem).wait()
        _compute_and_write(r0, fstage)

        @pl.when(g + 1 < HALF)
        def _():
            pltpu.async_copy(fshared.at[gidx.at[r0 + 2]], fstage, sem)

        pltpu.make_async_copy(fshared.at[gidx.at[r1]], fstage2, sem2).wait()
        _compute_and_write(r1, fstage2)
        return carry

    lax.fori_loop(0, HALF, out_group, 0)


def _sc_group(q_soa, src_soa, srcx32):
    mesh = plsc.VectorSubcoreMesh(core_axis_name="c", subcore_axis_name="s",
                                  num_cores=NC, num_subcores=NS)
    return pl.kernel(
        _sc_body,
        out_type=jax.ShapeDtypeStruct((K, Q, CPAD), jnp.float32),
        mesh=mesh,
        compiler_params=pltpu.CompilerParams(use_tc_tiling_on_sc=False,
                                             needs_layout_passes=False),
        scratch_types=[
            pltpu.VMEM((N,), jnp.float32),
            pltpu.VMEM((N,), jnp.float32),
            pltpu.VMEM((N,), jnp.float32),
            pltpu.VMEM((QPW,), jnp.float32),
            pltpu.VMEM((QPW,), jnp.float32),
            pltpu.VMEM((QPW,), jnp.float32),
            pltpu.VMEM((N,), jnp.float32),
            pltpu.VMEM((N,), jnp.float32),
            pltpu.VMEM((N,), jnp.float32),
            pltpu.VMEM((N,), jnp.float32),
            pltpu.VMEM((QPW * K,), jnp.int32),
            pltpu.VMEM((NGC, GCH), jnp.int32),
            pltpu.VMEM((GCH, CPAD), jnp.float32),
            pltpu.VMEM((GCH, CPAD), jnp.float32),
            pltpu.VMEM_SHARED((2 * N, CPAD), jnp.float32),
            pltpu.SemaphoreType.DMA,
            pltpu.SemaphoreType.DMA,
            pltpu.SemaphoreType.DMA,
        ],
    )(q_soa, src_soa, srcx32)


def _gelu(x):
    return 0.5 * x * (1.0 + lax.erf(x * jnp.float32(0.7071067811865475)))


def _t1_body(x_ref, w1p_ref, stats_ref):
    k = pl.program_id(0)
    y = lax.dot_general(x_ref[...].reshape(Q, CPAD), w1p_ref[...],
                        (((1,), (1,)), ((), ())),
                        preferred_element_type=jnp.float32)
    blk = jnp.stack([jnp.sum(y, 0), jnp.sum(y * y, 0)])

    @pl.when(k == 0)
    def _():
        stats_ref[...] = blk

    @pl.when(k > 0)
    def _():
        stats_ref[...] += blk


def _t2_body(x_ref, w1p_ref, a1_ref, c1_ref, w2_ref, zmax_ref, stats_ref):
    k = pl.program_id(0)
    y = lax.dot_general(x_ref[...].reshape(Q, CPAD), w1p_ref[...],
                        (((1,), (1,)), ((), ())),
                        preferred_element_type=jnp.float32)
    y = _gelu(y * a1_ref[...] + c1_ref[...])
    z = lax.dot_general(y, w2_ref[...], (((1,), (1,)), ((), ())),
                        preferred_element_type=jnp.float32)

    @pl.when(k == 0)
    def _():
        zmax_ref[...] = z

    @pl.when(k > 0)
    def _():
        zmax_ref[...] = jnp.maximum(zmax_ref[...], z)

    @pl.when(k == K - 1)
    def _():
        zm = zmax_ref[...]
        stats_ref[...] = jnp.stack([jnp.sum(zm, 0), jnp.sum(zm * zm, 0)])


def _t3_body(zmax_ref, a2_ref, c2_ref, out_ref):
    out_ref[...] = _gelu(zmax_ref[...] * a2_ref[...] + c2_ref[...]).reshape(
        B, M, COUT2)


def kernel(src_x, src_xyz, xyz, W1, g1, b1, W2, gl, bl):
    q_soa = xyz.reshape(Q, 3).T.reshape(3 * Q)
    src_soa = src_xyz.transpose(0, 2, 1).reshape(B * 3 * N)
    srcx32 = jnp.concatenate(
        [src_x.reshape(B * N, CIN),
         jnp.zeros((B * N, CPAD - CIN), jnp.float32)], axis=1)
    w1p = jnp.concatenate(
        [W1[:, 3:], W1[:, :3],
         jnp.zeros((COUT1, CPAD - CIN - 3), jnp.float32)], axis=1)

    x_g = _sc_group(q_soa, src_soa, srcx32)

    x_spec = pl.BlockSpec((1, Q, CPAD), lambda k: (k, 0, 0))
    full = lambda *shape: pl.BlockSpec(shape, lambda *_: (0,) * len(shape))

    stats1 = pl.pallas_call(
        _t1_body,
        grid=(K,),
        in_specs=[x_spec, full(COUT1, CPAD)],
        out_specs=full(2, COUT1),
        out_shape=jax.ShapeDtypeStruct((2, COUT1), jnp.float32),
    )(x_g, w1p)

    cnt1 = jnp.float32(Q * K)
    mu1 = stats1[0] / cnt1
    var1 = stats1[1] / cnt1 - mu1 * mu1
    a1 = (g1 / jnp.sqrt(var1 + EPS)).reshape(1, COUT1)
    c1 = (b1 - mu1 * a1[0]).reshape(1, COUT1)

    zmax, stats2 = pl.pallas_call(
        _t2_body,
        grid=(K,),
        in_specs=[x_spec, full(COUT1, CPAD),
                  full(1, COUT1), full(1, COUT1), full(COUT2, COUT1)],
        out_specs=[full(Q, COUT2), full(2, COUT2)],
        out_shape=[jax.ShapeDtypeStruct((Q, COUT2), jnp.float32),
                   jax.ShapeDtypeStruct((2, COUT2), jnp.float32)],
    )(x_g, w1p, a1, c1, W2)

    cnt2 = jnp.float32(Q)
    mu2 = stats2[0] / cnt2
    var2 = stats2[1] / cnt2 - mu2 * mu2
    a2 = (gl / jnp.sqrt(var2 + EPS)).reshape(1, COUT2)
    c2 = (bl - mu2 * a2[0]).reshape(1, COUT2)

    out = pl.pallas_call(
        _t3_body,
        grid=(1,),
        in_specs=[full(Q, COUT2), full(1, COUT2), full(1, COUT2)],
        out_specs=full(B, M, COUT2),
        out_shape=jax.ShapeDtypeStruct((B, M, COUT2), jnp.float32),
    )(zmax, a2, c2)
    return out
